# baseline (device time: 35145 ns/iter reference)
import jax
import jax.numpy as jnp
from jax import lax
from jax.experimental import pallas as pl
from jax.experimental.pallas import tpu as pltpu

N_DEV = 16
N_TOK = 512
D_IN = 256
D_OUT = 512
N_EXP = 64
N_EXP_LOCAL = N_EXP // N_DEV
CHUNK = N_TOK // N_DEV


def kernel(x, router_W, route_idx, expert_W):
    def body(
        x_ref,
        rw_ref,
        idx_ref,
        ew_ref,
        out_ref,
        partial_ref,
        red_ref,
        rs_buf,
        ag_buf,
        rs_send_sems,
        rs_recv_sems,
        ag_send_sems,
        ag_recv_sems,
    ):
        my = lax.axis_index("i")

        barrier = pltpu.get_barrier_semaphore()
        for d in range(1, N_DEV):
            peer = jnp.remainder(my + d, N_DEV)
            pl.semaphore_signal(
                barrier,
                inc=1,
                device_id=(peer,),
                device_id_type=pl.DeviceIdType.MESH,
            )
        pl.semaphore_wait(barrier, N_DEV - 1)

        xf = x_ref[...]
        scores = jnp.dot(xf, rw_ref[...], preferred_element_type=jnp.float32)
        smax = jnp.max(scores, axis=-1, keepdims=True)
        p = jnp.exp(scores - smax)
        p = p / jnp.sum(p, axis=-1, keepdims=True)

        idx0 = idx_ref[...][:, 0:1]
        idx1 = idx_ref[...][:, 1:2]
        eiota = lax.broadcasted_iota(jnp.int32, (N_TOK, N_EXP), 1)
        g0 = jnp.sum(
            jnp.where(eiota == idx0, p, 0.0), axis=1, keepdims=True
        )
        g1 = jnp.sum(
            jnp.where(eiota == idx1, p, 0.0), axis=1, keepdims=True
        )
        gs = g0 + g1
        w0 = g0 / gs
        w1 = g1 / gs

        acc = jnp.zeros((N_TOK, D_OUT), jnp.float32)
        for e in range(N_EXP_LOCAL):
            eg = my * N_EXP_LOCAL + e
            gate = jnp.where(idx0 == eg, w0, 0.0) + jnp.where(
                idx1 == eg, w1, 0.0
            )
            xg = (xf * gate).astype(jnp.bfloat16)
            acc = acc + jnp.dot(
                xg,
                ew_ref[e].astype(jnp.bfloat16),
                preferred_element_type=jnp.float32,
            )
        partial_ref[...] = acc

        rs = []
        for d in range(1, N_DEV):
            peer = jnp.remainder(my + d, N_DEV)
            desc = pltpu.make_async_remote_copy(
                src_ref=partial_ref.at[pl.ds(peer * CHUNK, CHUNK), :],
                dst_ref=rs_buf.at[d],
                send_sem=rs_send_sems.at[d],
                recv_sem=rs_recv_sems.at[d],
                device_id=(peer,),
                device_id_type=pl.DeviceIdType.MESH,
            )
            desc.start()
            rs.append(desc)

        for desc in rs:
            desc.wait_recv()

        red = partial_ref[pl.ds(my * CHUNK, CHUNK), :]
        for d in range(1, N_DEV):
            red = red + rs_buf[d]
        red_ref[...] = red

        ag = []
        for d in range(1, N_DEV):
            peer = jnp.remainder(my + d, N_DEV)
            desc = pltpu.make_async_remote_copy(
                src_ref=red_ref,
                dst_ref=ag_buf.at[d],
                send_sem=ag_send_sems.at[d],
                recv_sem=ag_recv_sems.at[d],
                device_id=(peer,),
                device_id_type=pl.DeviceIdType.MESH,
            )
            desc.start()
            ag.append(desc)

        out_ref[pl.ds(my * CHUNK, CHUNK), :] = red
        for d in range(1, N_DEV):
            ag[d - 1].wait_recv()
            src_dev_chunk = jnp.remainder(my - d, N_DEV)
            out_ref[pl.ds(src_dev_chunk * CHUNK, CHUNK), :] = ag_buf[d]

        for desc in rs:
            desc.wait_send()
        for desc in ag:
            desc.wait_send()

    return pl.pallas_call(
        body,
        out_shape=jax.ShapeDtypeStruct((N_TOK, D_OUT), jnp.float32),
        in_specs=[
            pl.BlockSpec(memory_space=pltpu.VMEM),
            pl.BlockSpec(memory_space=pltpu.VMEM),
            pl.BlockSpec(memory_space=pltpu.VMEM),
            pl.BlockSpec(memory_space=pltpu.VMEM),
        ],
        out_specs=pl.BlockSpec(memory_space=pltpu.VMEM),
        scratch_shapes=[
            pltpu.VMEM((N_TOK, D_OUT), jnp.float32),
            pltpu.VMEM((CHUNK, D_OUT), jnp.float32),
            pltpu.VMEM((N_DEV, CHUNK, D_OUT), jnp.float32),
            pltpu.VMEM((N_DEV, CHUNK, D_OUT), jnp.float32),
            pltpu.SemaphoreType.DMA((N_DEV,)),
            pltpu.SemaphoreType.DMA((N_DEV,)),
            pltpu.SemaphoreType.DMA((N_DEV,)),
            pltpu.SemaphoreType.DMA((N_DEV,)),
        ],
        compiler_params=pltpu.CompilerParams(collective_id=0),
    )(x, router_W, route_idx, expert_W)


# device time: 26840 ns/iter; 1.3094x vs baseline; 1.3094x over previous
import jax
import jax.numpy as jnp
from jax import lax
from jax.experimental import pallas as pl
from jax.experimental.pallas import tpu as pltpu

N_DEV = 16
N_TOK = 512
D_IN = 256
D_OUT = 512
N_EXP = 64
N_EXP_LOCAL = N_EXP // N_DEV
CHUNK = N_TOK // N_DEV


def kernel(x, router_W, route_idx, expert_W):
    def body(
        x_ref,
        rw_ref,
        idx_ref,
        ew_ref,
        out_ref,
        partial_ref,
        red_ref,
        rs_buf,
        ag_buf,
        rs_send_sems,
        rs_recv_sems,
        ag_send_sems,
        ag_recv_sems,
    ):
        my = lax.axis_index("i")

        barrier = pltpu.get_barrier_semaphore()
        for d in range(1, N_DEV):
            peer = jnp.remainder(my + d, N_DEV)
            pl.semaphore_signal(
                barrier,
                inc=1,
                device_id=(peer,),
                device_id_type=pl.DeviceIdType.MESH,
            )
        pl.semaphore_wait(barrier, N_DEV - 1)

        xf = x_ref[...]
        scores = jnp.dot(xf, rw_ref[...], preferred_element_type=jnp.float32)
        smax = jnp.max(scores, axis=-1, keepdims=True)
        p = jnp.exp(scores - smax)
        p = p / jnp.sum(p, axis=-1, keepdims=True)

        idx0 = idx_ref[...][:, 0:1]
        idx1 = idx_ref[...][:, 1:2]
        eiota = lax.broadcasted_iota(jnp.int32, (N_TOK, N_EXP), 1)
        g0 = jnp.sum(
            jnp.where(eiota == idx0, p, 0.0), axis=1, keepdims=True
        )
        g1 = jnp.sum(
            jnp.where(eiota == idx1, p, 0.0), axis=1, keepdims=True
        )
        gs = g0 + g1
        w0 = g0 / gs
        w1 = g1 / gs

        acc = jnp.zeros((N_TOK, D_OUT), jnp.float32)
        for e in range(N_EXP_LOCAL):
            eg = my * N_EXP_LOCAL + e
            gate = jnp.where(idx0 == eg, w0, 0.0) + jnp.where(
                idx1 == eg, w1, 0.0
            )
            xg = (xf * gate).astype(jnp.bfloat16)
            acc = acc + jnp.dot(
                xg,
                ew_ref[e].astype(jnp.bfloat16),
                preferred_element_type=jnp.float32,
            )
        partial_ref[...] = acc.astype(jnp.bfloat16)

        rs = []
        for d in range(1, N_DEV):
            peer = jnp.remainder(my + d, N_DEV)
            desc = pltpu.make_async_remote_copy(
                src_ref=partial_ref.at[pl.ds(peer * CHUNK, CHUNK), :],
                dst_ref=rs_buf.at[d],
                send_sem=rs_send_sems.at[d],
                recv_sem=rs_recv_sems.at[d],
                device_id=(peer,),
                device_id_type=pl.DeviceIdType.MESH,
            )
            desc.start()
            rs.append(desc)

        red = partial_ref[pl.ds(my * CHUNK, CHUNK), :].astype(jnp.float32)
        for d in range(1, N_DEV):
            rs[d - 1].wait_recv()
            red = red + rs_buf[d].astype(jnp.float32)
        red_ref[...] = red.astype(jnp.bfloat16)

        ag = []
        for d in range(1, N_DEV):
            peer = jnp.remainder(my + d, N_DEV)
            desc = pltpu.make_async_remote_copy(
                src_ref=red_ref,
                dst_ref=ag_buf.at[d],
                send_sem=ag_send_sems.at[d],
                recv_sem=ag_recv_sems.at[d],
                device_id=(peer,),
                device_id_type=pl.DeviceIdType.MESH,
            )
            desc.start()
            ag.append(desc)

        out_ref[pl.ds(my * CHUNK, CHUNK), :] = red
        for d in range(1, N_DEV):
            ag[d - 1].wait_recv()
            src_dev_chunk = jnp.remainder(my - d, N_DEV)
            out_ref[pl.ds(src_dev_chunk * CHUNK, CHUNK), :] = ag_buf[
                d
            ].astype(jnp.float32)

        for desc in rs:
            desc.wait_send()
        for desc in ag:
            desc.wait_send()

    return pl.pallas_call(
        body,
        out_shape=jax.ShapeDtypeStruct((N_TOK, D_OUT), jnp.float32),
        in_specs=[
            pl.BlockSpec(memory_space=pltpu.VMEM),
            pl.BlockSpec(memory_space=pltpu.VMEM),
            pl.BlockSpec(memory_space=pltpu.VMEM),
            pl.BlockSpec(memory_space=pltpu.VMEM),
        ],
        out_specs=pl.BlockSpec(memory_space=pltpu.VMEM),
        scratch_shapes=[
            pltpu.VMEM((N_TOK, D_OUT), jnp.bfloat16),
            pltpu.VMEM((CHUNK, D_OUT), jnp.bfloat16),
            pltpu.VMEM((N_DEV, CHUNK, D_OUT), jnp.bfloat16),
            pltpu.VMEM((N_DEV, CHUNK, D_OUT), jnp.bfloat16),
            pltpu.SemaphoreType.DMA((N_DEV,)),
            pltpu.SemaphoreType.DMA((N_DEV,)),
            pltpu.SemaphoreType.DMA((N_DEV,)),
            pltpu.SemaphoreType.DMA((N_DEV,)),
        ],
        compiler_params=pltpu.CompilerParams(collective_id=0),
    )(x, router_W, route_idx, expert_W)
